# baseline (speedup 1.0000x reference)
import jax
import jax.numpy as jnp
from jax.experimental import pallas as pl


def _rowdot_kernel(gu_ref, gi_ref, out_ref):
    out_ref[:] = jnp.sum(gu_ref[:] * gi_ref[:], axis=1)


def kernel(gu, gi):
    B, D = gu.shape
    BLK = 2048
    xui = pl.pallas_call(
        _rowdot_kernel,
        grid=(B // BLK,),
        in_specs=[
            pl.BlockSpec((BLK, D), lambda i: (i, 0)),
            pl.BlockSpec((BLK, D), lambda i: (i, 0)),
        ],
        out_specs=pl.BlockSpec((BLK,), lambda i: (i,)),
        out_shape=jax.ShapeDtypeStruct((B,), jnp.float32),
    )(gu, gi)
    return (xui, gu, gi)
